# baseline (speedup 1.0000x reference)
import functools

import jax
import jax.numpy as jnp
from jax import lax
from jax.experimental import pallas as pl
from jax.experimental.pallas import tpu as pltpu
from jax.experimental.pallas import tpu_sc as plsc

_B = 16384
_D = 64
_NC = 2
_NS = 16
_NW = _NC * _NS
_BW = _B // _NW
_K = 128
_NCHUNK = _BW // _K


def _dist_body(trip_hbm, table_hbm, out_hbm,
               trip_v, off_s_sm, off_d_sm,
               rows_s0, rows_d0, rows_s1, rows_d1, part_v, out_v,
               sem0, sem1):
    wid = lax.axis_index("s") * _NC + lax.axis_index("c")
    base = wid * _BW

    pltpu.sync_copy(trip_hbm.at[:, pl.ds(base, _BW)], trip_v)

    lane16 = lax.iota(jnp.int32, 16) * 16
    rows = ((rows_s0, rows_d0), (rows_s1, rows_d1))
    sems = (sem0, sem1)

    def issue(chunk, rs, rd, sem):
        off = chunk * _K

        def body(i16, _):
            i0 = off + i16 * 16
            vs = trip_v[0, pl.ds(i0, 16)]
            vd = trip_v[1, pl.ds(i0, 16)]
            for j in range(16):
                r = vs[j]
                pltpu.async_copy(table_hbm.at[r >> 1],
                                 rs.at[i16 * 16 + j], sem)
                off_s_sm[i0 + j] = (r & 1) * _D
                r2 = vd[j]
                pltpu.async_copy(table_hbm.at[r2 >> 1],
                                 rd.at[i16 * 16 + j], sem)
                off_d_sm[i0 + j] = (r2 & 1) * _D
            return 0

        lax.fori_loop(0, _K // 16, body, 0)

    def drain(rs, rd, sem):
        def body(i, _):
            pltpu.make_async_copy(table_hbm.at[0], rs.at[0], sem).wait()
            pltpu.make_async_copy(table_hbm.at[0], rd.at[0], sem).wait()
            return 0

        lax.fori_loop(0, _K, body, 0)

    def compute(chunk, rs, rd):
        off = chunk * _K

        def group(g, _):
            for rloc in range(16):
                row = g * 16 + rloc
                o_s = off_s_sm[off + row]
                o_d = off_d_sm[off + row]
                p = jnp.zeros((16,), jnp.float32)
                for c in range(0, _D, 16):
                    s = rs[row, pl.ds(o_s + c, 16)]
                    t = rd[row, pl.ds(o_d + c, 16)]
                    df = s - t
                    p = p + df * df
                part_v[pl.ds(rloc * 16, 16)] = p
            acc = jnp.full((16,), 1e-12, jnp.float32)
            for k in range(16):
                acc = acc + plsc.load_gather(part_v, [lane16 + k])
            yi = 0x5F3759DF - (plsc.bitcast(acc, jnp.int32) >> 1)
            y = plsc.bitcast(yi, jnp.float32)
            y = y * (1.5 - 0.5 * acc * y * y)
            y = y * (1.5 - 0.5 * acc * y * y)
            y = y * (1.5 - 0.5 * acc * y * y)
            out_v[pl.ds(off + g * 16, 16)] = acc * y
            return 0

        lax.fori_loop(0, _K // 16, group, 0)

    issue(0, rows[0][0], rows[0][1], sems[0])
    for c in range(_NCHUNK):
        p = c % 2
        if c + 1 < _NCHUNK:
            np_ = (c + 1) % 2
            issue(c + 1, rows[np_][0], rows[np_][1], sems[np_])
        drain(rows[p][0], rows[p][1], sems[p])
        compute(c, rows[p][0], rows[p][1])

    pltpu.sync_copy(out_v, out_hbm.at[pl.ds(base, _BW)])


_dist_kernel = functools.partial(
    pl.kernel,
    mesh=plsc.VectorSubcoreMesh(core_axis_name="c", subcore_axis_name="s"),
    out_type=jax.ShapeDtypeStruct((_B,), jnp.float32),
    compiler_params=pltpu.CompilerParams(needs_layout_passes=False),
    scratch_types=[
        pltpu.VMEM((3, _BW), jnp.int32),
        pltpu.SMEM((_BW,), jnp.int32),
        pltpu.SMEM((_BW,), jnp.int32),
        pltpu.VMEM((_K, 2 * _D), jnp.float32),
        pltpu.VMEM((_K, 2 * _D), jnp.float32),
        pltpu.VMEM((_K, 2 * _D), jnp.float32),
        pltpu.VMEM((_K, 2 * _D), jnp.float32),
        pltpu.VMEM((256,), jnp.float32),
        pltpu.VMEM((_BW,), jnp.float32),
        pltpu.SemaphoreType.DMA,
        pltpu.SemaphoreType.DMA,
    ],
)(_dist_body)


def kernel(input_triplet, embeds):
    trip = input_triplet.T
    table2 = embeds.reshape(500000, 2 * _D)
    return _dist_kernel(trip, table2)


# speedup vs baseline: 2.5254x; 2.5254x over previous
import functools

import jax
import jax.numpy as jnp
from jax import lax
from jax.experimental import pallas as pl
from jax.experimental.pallas import tpu as pltpu
from jax.experimental.pallas import tpu_sc as plsc

_B = 16384
_D = 64
_NC = 2
_NS = 16
_NW = _NC * _NS
_BW = _B // _NW
_K = 256
_NCHUNK = _BW // _K


def _dist_body(trip_hbm, table_hbm, out_hbm,
               trip_v, rows_s, rows_d, part_v, out_v, sem):
    wid = lax.axis_index("s") * _NC + lax.axis_index("c")
    base = wid * _BW

    pltpu.sync_copy(trip_hbm.at[:, pl.ds(base, _BW)], trip_v)

    lane16 = lax.iota(jnp.int32, 16) * 16

    for chunk in range(_NCHUNK):
        off = chunk * _K

        def issue(i16, _):
            vs = trip_v[0, pl.ds(off + i16 * 16, 16)]
            vd = trip_v[1, pl.ds(off + i16 * 16, 16)]
            for j in range(16):
                r = vs[j]
                pltpu.async_copy(table_hbm.at[r >> 3, r & 7],
                                 rows_s.at[i16 * 16 + j], sem)
                r2 = vd[j]
                pltpu.async_copy(table_hbm.at[r2 >> 3, r2 & 7],
                                 rows_d.at[i16 * 16 + j], sem)
            return 0

        lax.fori_loop(0, _K // 16, issue, 0)

        def drain(i, _):
            pltpu.make_async_copy(table_hbm.at[0, 0], rows_s.at[0], sem).wait()
            pltpu.make_async_copy(table_hbm.at[0, 0], rows_d.at[0], sem).wait()
            return 0

        lax.fori_loop(0, _K, drain, 0)

        def group(g, _):
            for rloc in range(16):
                row = g * 16 + rloc
                p = jnp.zeros((16,), jnp.float32)
                for c in range(0, _D, 16):
                    s = rows_s[row, pl.ds(c, 16)]
                    t = rows_d[row, pl.ds(c, 16)]
                    df = s - t
                    p = p + df * df
                part_v[pl.ds(rloc * 16, 16)] = p
            acc = jnp.full((16,), 1e-12, jnp.float32)
            for k in range(16):
                acc = acc + plsc.load_gather(part_v, [lane16 + k])
            yi = 0x5F3759DF - (plsc.bitcast(acc, jnp.int32) >> 1)
            y = plsc.bitcast(yi, jnp.float32)
            y = y * (1.5 - 0.5 * acc * y * y)
            y = y * (1.5 - 0.5 * acc * y * y)
            y = y * (1.5 - 0.5 * acc * y * y)
            out_v[pl.ds(off + g * 16, 16)] = acc * y
            return 0

        lax.fori_loop(0, _K // 16, group, 0)

    pltpu.sync_copy(out_v, out_hbm.at[pl.ds(base, _BW)])


_dist_kernel = functools.partial(
    pl.kernel,
    mesh=plsc.VectorSubcoreMesh(core_axis_name="c", subcore_axis_name="s"),
    out_type=jax.ShapeDtypeStruct((_B,), jnp.float32),
    compiler_params=pltpu.CompilerParams(needs_layout_passes=False),
    scratch_types=[
        pltpu.VMEM((3, _BW), jnp.int32),
        pltpu.VMEM((_K, _D), jnp.float32),
        pltpu.VMEM((_K, _D), jnp.float32),
        pltpu.VMEM((256,), jnp.float32),
        pltpu.VMEM((_BW,), jnp.float32),
        pltpu.SemaphoreType.DMA,
    ],
)(_dist_body)


def kernel(input_triplet, embeds):
    trip = input_triplet.T
    table3 = embeds.reshape(125000, 8, _D)
    return _dist_kernel(trip, table3)
